# baseline (speedup 1.0000x reference)
import functools

import jax
import jax.numpy as jnp
import numpy as np
from jax import lax
from jax.experimental import pallas as pl
from jax.experimental.pallas import tpu as pltpu
from jax.experimental.pallas import tpu_sc as plsc

N_NODES = 10000
P_NODES = 10240
N_EDGES = 320000
N_WORKERS = 32
D_FEAT = 128
EPW = 10240
P_EDGES = N_WORKERS * EPW
STATE = 128
N_CLASSES = 40
NUM_LAYERS = 4
BOUND = 10
BN_SCALE = np.float32(1.0 / np.sqrt(1.0 + 1e-5))
BLK = 1024
GRID = P_NODES // BLK
HALF = P_NODES // 2
NSH = HALF * STATE


def _argmax_soft(l):
    m = jnp.max(l, axis=-1, keepdims=True)
    p = jnp.exp(l - m)
    q = p / jnp.sum(p, axis=-1, keepdims=True)
    qm = jnp.max(q, axis=-1, keepdims=True)
    ii = lax.broadcasted_iota(jnp.int32, l.shape, l.ndim - 1)
    return jnp.min(jnp.where(q == qm, ii, STATE), axis=-1)


def _input_body(x_ref, w_ref, b_ref, out_ref):
    l = jnp.dot(x_ref[...], w_ref[...], preferred_element_type=jnp.float32)
    out_ref[...] = _argmax_soft(l + b_ref[...])


def _input_call(x_pad, w_in, b_in):
    return pl.pallas_call(
        _input_body,
        grid=(GRID,),
        in_specs=[
            pl.BlockSpec((BLK, STATE), lambda i: (i, 0)),
            pl.BlockSpec((STATE, STATE), lambda i: (0, 0)),
            pl.BlockSpec((1, STATE), lambda i: (0, 0)),
        ],
        out_specs=pl.BlockSpec((BLK,), lambda i: (i,)),
        out_shape=jax.ShapeDtypeStruct((P_NODES,), jnp.int32),
    )(x_pad, w_in, b_in)


def _table_body(w_ref, b_ref, out_ref):
    i = pl.program_id(0)
    p = i * BLK + lax.broadcasted_iota(jnp.int32, (BLK, 2 * STATE), 0)
    ii = lax.broadcasted_iota(jnp.int32, (BLK, 2 * STATE), 1)
    a = p // STATE
    b = p % STATE
    e = ((ii == a) | (ii == b + STATE)).astype(jnp.float32)
    l = (jnp.dot(e, w_ref[...], preferred_element_type=jnp.float32)
         + b_ref[...]) * BN_SCALE
    out_ref[...] = _argmax_soft(l)


def _table_call(w_edge, b_edge):
    return pl.pallas_call(
        _table_body,
        grid=(STATE * STATE // BLK,),
        in_specs=[
            pl.BlockSpec((2 * STATE, STATE), lambda i: (0, 0)),
            pl.BlockSpec((1, STATE), lambda i: (0, 0)),
        ],
        out_specs=pl.BlockSpec((BLK,), lambda i: (i,)),
        out_shape=jax.ShapeDtypeStruct((STATE * STATE,), jnp.int32),
    )(w_edge, b_edge.reshape(1, STATE))


def _new_h(c0_ref, c1_ref, h_ref, w_ref, b_ref, eps_ref):
    packed = c0_ref[0] + c1_ref[0]
    lo = jnp.bitwise_and(packed, 0xFFFF)
    hi = lax.shift_right_logical(packed, 16)
    cnt = jnp.where(pl.program_id(0) < GRID // 2, lo, hi)
    aggr = jnp.minimum(cnt, BOUND).astype(jnp.float32)
    ii = lax.broadcasted_iota(jnp.int32, aggr.shape, 1)
    hcol = h_ref[...][:, None]
    a = aggr + jnp.where(ii == hcol, 1.0 + eps_ref[0, 0], 0.0)
    l = (jnp.dot(a, w_ref[...], preferred_element_type=jnp.float32)
         + b_ref[...]) * BN_SCALE
    return _argmax_soft(l)


def _update_body(c0_ref, c1_ref, h_ref, w_ref, b_ref, eps_ref, out_ref):
    out_ref[...] = _new_h(c0_ref, c1_ref, h_ref, w_ref, b_ref, eps_ref)


_COUNT_SPECS = [
    pl.BlockSpec((1, BLK, STATE), lambda i: (0, i % (GRID // 2), 0)),
    pl.BlockSpec((1, BLK, STATE), lambda i: (1, i % (GRID // 2), 0)),
    pl.BlockSpec((BLK,), lambda i: (i,)),
    pl.BlockSpec((STATE, STATE), lambda i: (0, 0)),
    pl.BlockSpec((1, STATE), lambda i: (0, 0)),
    pl.BlockSpec((1, 1), lambda i: (0, 0)),
]


def _update_call(counts3, h_idx, w_node, b_node, eps2):
    return pl.pallas_call(
        _update_body,
        grid=(GRID,),
        in_specs=_COUNT_SPECS,
        out_specs=pl.BlockSpec((BLK,), lambda i: (i,)),
        out_shape=jax.ShapeDtypeStruct((P_NODES,), jnp.int32),
    )(counts3, counts3, h_idx, w_node, b_node, eps2)


def _final_body(c0_ref, c1_ref, h_ref, w_ref, b_ref, eps_ref,
                wp_ref, bp_ref, out_ref):
    idx = _new_h(c0_ref, c1_ref, h_ref, w_ref, b_ref, eps_ref)
    ii = lax.broadcasted_iota(jnp.int32, (BLK, STATE), 1)
    oh = jnp.where(ii == idx[:, None], 1.0, 0.0)
    out_ref[...] = jnp.dot(oh, wp_ref[...], preferred_element_type=jnp.float32,
                           precision=lax.Precision.HIGHEST) + bp_ref[...]


def _final_call(counts3, h_idx, w_node, b_node, eps2, w_pool, b_pool):
    return pl.pallas_call(
        _final_body,
        grid=(GRID,),
        in_specs=_COUNT_SPECS + [
            pl.BlockSpec((STATE, N_CLASSES), lambda i: (0, 0)),
            pl.BlockSpec((1, N_CLASSES), lambda i: (0, 0)),
        ],
        out_specs=pl.BlockSpec((BLK, N_CLASSES), lambda i: (i, 0)),
        out_shape=jax.ShapeDtypeStruct((P_NODES, N_CLASSES), jnp.float32),
    )(counts3, counts3, h_idx, w_node, b_node, eps2, w_pool, b_pool)


PART = 2048
NPART = EPW // PART
ZCH = 2560


def _sc_body(hidx_hbm, src_hbm, dst_hbm, t_hbm, out_hbm,
             src_v, dst_v, hidx_v, t_v, keys_v, vals_v, zb_v,
             counts_sh, sc_sem):
    cc = lax.axis_index("c")
    ss = lax.axis_index("s")

    zeros16 = jnp.zeros((16,), jnp.int32)

    def fill_z(i, carry):
        zb_v[pl.ds(i * 16, 16)] = zeros16
        return carry
    lax.fori_loop(0, ZCH // 16, fill_z, 0)

    nz = NSH // 16 // ZCH
    def zero_c(i, carry):
        pltpu.async_copy(zb_v,
                         counts_sh.at[pl.ds((ss * nz + i) * ZCH, ZCH)],
                         sc_sem)
        return carry
    lax.fori_loop(0, nz, zero_c, 0)

    base = cc * (P_EDGES // 2) + ss * EPW
    pltpu.sync_copy(hidx_hbm, hidx_v)
    pltpu.sync_copy(t_hbm, t_v)

    def zero_drain(i, carry):
        pltpu.make_async_copy(
            zb_v, counts_sh.at[pl.ds((ss * nz + i) * ZCH, ZCH)],
            sc_sem).wait()
        return carry
    lax.fori_loop(0, nz, zero_drain, 0)
    plsc.subcore_barrier()

    def part(p, carry):
        pb = base + p * PART
        pltpu.sync_copy(src_hbm.at[pl.ds(pb, PART)], src_v)
        pltpu.sync_copy(dst_hbm.at[pl.ds(pb, PART)], dst_v)

        def chunk(s, c2):
            for j in range(8):
                off = s * 128 + j * 16
                sv = src_v[pl.ds(off, 16)]
                dv = dst_v[pl.ds(off, 16)]
                spos = (sv >> 1) + (sv & 1) * HALF
                dpos = (dv >> 1) + (dv & 1) * HALF
                a = plsc.load_gather(hidx_v, [spos])
                b = plsc.load_gather(hidx_v, [dpos])
                cidx = plsc.load_gather(t_v, [a * STATE + b])
                keys_v[s, pl.ds(j * 16, 16)] = (dv >> 1) * STATE + cidx
                vals_v[s, pl.ds(j * 16, 16)] = jnp.left_shift(
                    1, (dv & 1) * 16)
            pltpu.async_copy(vals_v.at[s], counts_sh.at[keys_v.at[s]],
                             sc_sem, add=True)
            return c2
        lax.fori_loop(0, PART // 128, chunk, 0)

        def drain(s, c2):
            pltpu.make_async_copy(vals_v.at[s], counts_sh.at[keys_v.at[s]],
                                  sc_sem).wait()
            return c2
        lax.fori_loop(0, PART // 128, drain, 0)
        return carry
    lax.fori_loop(0, NPART, part, 0)
    plsc.subcore_barrier()

    och = NSH // 16
    pltpu.sync_copy(counts_sh.at[pl.ds(ss * och, och)],
                    out_hbm.at[pl.ds(cc * NSH + ss * och, och)])


@functools.lru_cache(maxsize=None)
def _get_sc_histogram():
    mesh = plsc.VectorSubcoreMesh(core_axis_name="c", subcore_axis_name="s")
    return pl.kernel(
        _sc_body,
        out_type=jax.ShapeDtypeStruct((2 * NSH,), jnp.int32),
        mesh=mesh,
        compiler_params=pltpu.CompilerParams(needs_layout_passes=False),
        scratch_types=[
            pltpu.VMEM((PART,), jnp.int32),
            pltpu.VMEM((PART,), jnp.int32),
            pltpu.VMEM((P_NODES,), jnp.int32),
            pltpu.VMEM((STATE * STATE,), jnp.int32),
            pltpu.VMEM((PART // 128, 128), jnp.int32),
            pltpu.VMEM((PART // 128, 128), jnp.int32),
            pltpu.VMEM((ZCH,), jnp.int32),
            pltpu.VMEM_SHARED((NSH,), jnp.int32),
            pltpu.SemaphoreType.DMA,
        ],
    )


def kernel(x, edge_index, W_in, b_in, W_edge, b_edge, W_node, b_node, eps,
           W_pool, b_pool):
    src = edge_index[0].astype(jnp.int32)
    dst = edge_index[1].astype(jnp.int32)
    pad = P_EDGES - N_EDGES
    srcp = jnp.concatenate([src, jnp.zeros((pad,), jnp.int32)])
    dstp = jnp.concatenate([dst, jnp.full((pad,), N_NODES, jnp.int32)])
    x_pad = jnp.concatenate(
        [x, jnp.zeros((P_NODES - N_NODES, x.shape[1]), x.dtype)])
    xp3 = x_pad.reshape(HALF, 2, D_FEAT)
    x_pad = jnp.concatenate([xp3[:, 0, :], xp3[:, 1, :]])
    b_in2 = b_in.reshape(1, STATE)
    b_node2 = b_node.reshape(1, STATE)
    b_pool2 = b_pool.reshape(1, N_CLASSES)
    eps2 = eps.reshape(1, 1).astype(jnp.float32)

    h_idx = _input_call(x_pad, W_in, b_in2)
    t_flat = _table_call(W_edge, b_edge)

    sc_hist = _get_sc_histogram()
    for layer in range(NUM_LAYERS):
        counts = sc_hist(h_idx, srcp, dstp, t_flat)
        counts3 = counts.reshape(2, HALF, STATE)
        if layer < NUM_LAYERS - 1:
            h_idx = _update_call(counts3, h_idx, W_node, b_node2, eps2)
        else:
            out = _final_call(counts3, h_idx, W_node, b_node2, eps2,
                              W_pool, b_pool2)
    out_il = jnp.stack([out[:HALF], out[HALF:]], axis=1)
    return out_il.reshape(P_NODES, N_CLASSES)[:N_NODES]


# speedup vs baseline: 1.0206x; 1.0206x over previous
import functools

import jax
import jax.numpy as jnp
import numpy as np
from jax import lax
from jax.experimental import pallas as pl
from jax.experimental.pallas import tpu as pltpu
from jax.experimental.pallas import tpu_sc as plsc

N_NODES = 10000
P_NODES = 10240
N_EDGES = 320000
N_WORKERS = 32
EPW = 10240
P_EDGES = N_WORKERS * EPW
STATE = 128
N_CLASSES = 40
NUM_LAYERS = 4
BOUND = 10
BN_SCALE = np.float32(1.0 / np.sqrt(1.0 + 1e-5))
BLK = 1024
GRID = P_NODES // BLK
HALF = P_NODES // 2
NSH = HALF * STATE
PERM = np.concatenate([np.arange(0, P_NODES, 2), np.arange(1, P_NODES, 2)])
UNPERM = np.empty((P_NODES,), np.int32)
UNPERM[PERM] = np.arange(P_NODES)


def _argmax_soft(l):
    m = jnp.max(l, axis=-1, keepdims=True)
    p = jnp.exp(l - m)
    q = p / jnp.sum(p, axis=-1, keepdims=True)
    qm = jnp.max(q, axis=-1, keepdims=True)
    ii = lax.broadcasted_iota(jnp.int32, l.shape, l.ndim - 1)
    return jnp.min(jnp.where(q == qm, ii, STATE), axis=-1)


def _input_body(x_ref, w_ref, b_ref, out_ref):
    l = jnp.dot(x_ref[...], w_ref[...], preferred_element_type=jnp.float32)
    out_ref[...] = _argmax_soft(l + b_ref[...])


def _input_call(x_pad, w_in, b_in):
    return pl.pallas_call(
        _input_body,
        grid=(GRID,),
        in_specs=[
            pl.BlockSpec((BLK, STATE), lambda i: (i, 0)),
            pl.BlockSpec((STATE, STATE), lambda i: (0, 0)),
            pl.BlockSpec((1, STATE), lambda i: (0, 0)),
        ],
        out_specs=pl.BlockSpec((BLK,), lambda i: (i,)),
        out_shape=jax.ShapeDtypeStruct((P_NODES,), jnp.int32),
    )(x_pad, w_in, b_in)


def _table_body(w_ref, b_ref, out_ref):
    i = pl.program_id(0)
    p = i * BLK + lax.broadcasted_iota(jnp.int32, (BLK, 2 * STATE), 0)
    ii = lax.broadcasted_iota(jnp.int32, (BLK, 2 * STATE), 1)
    a = p // STATE
    b = p % STATE
    e = ((ii == a) | (ii == b + STATE)).astype(jnp.float32)
    l = (jnp.dot(e, w_ref[...], preferred_element_type=jnp.float32)
         + b_ref[...]) * BN_SCALE
    out_ref[...] = _argmax_soft(l)


def _table_call(w_edge, b_edge):
    return pl.pallas_call(
        _table_body,
        grid=(STATE * STATE // BLK,),
        in_specs=[
            pl.BlockSpec((2 * STATE, STATE), lambda i: (0, 0)),
            pl.BlockSpec((1, STATE), lambda i: (0, 0)),
        ],
        out_specs=pl.BlockSpec((BLK,), lambda i: (i,)),
        out_shape=jax.ShapeDtypeStruct((STATE * STATE,), jnp.int32),
    )(w_edge, b_edge.reshape(1, STATE))


def _new_h(c0_ref, c1_ref, h_ref, w_ref, b_ref, eps_ref):
    packed = c0_ref[0] + c1_ref[0]
    lo = jnp.bitwise_and(packed, 0xFFFF)
    hi = lax.shift_right_logical(packed, 16)
    cnt = jnp.where(pl.program_id(0) < GRID // 2, lo, hi)
    aggr = jnp.minimum(cnt, BOUND).astype(jnp.float32)
    ii = lax.broadcasted_iota(jnp.int32, aggr.shape, 1)
    hcol = h_ref[...][:, None]
    a = aggr + jnp.where(ii == hcol, 1.0 + eps_ref[0, 0], 0.0)
    l = (jnp.dot(a, w_ref[...], preferred_element_type=jnp.float32)
         + b_ref[...]) * BN_SCALE
    return _argmax_soft(l)


def _update_body(c0_ref, c1_ref, h_ref, w_ref, b_ref, eps_ref, out_ref):
    out_ref[...] = _new_h(c0_ref, c1_ref, h_ref, w_ref, b_ref, eps_ref)


_COUNT_SPECS = [
    pl.BlockSpec((1, BLK, STATE), lambda i: (0, i % (GRID // 2), 0)),
    pl.BlockSpec((1, BLK, STATE), lambda i: (1, i % (GRID // 2), 0)),
    pl.BlockSpec((BLK,), lambda i: (i,)),
    pl.BlockSpec((STATE, STATE), lambda i: (0, 0)),
    pl.BlockSpec((1, STATE), lambda i: (0, 0)),
    pl.BlockSpec((1, 1), lambda i: (0, 0)),
]


def _update_call(counts3, h_idx, w_node, b_node, eps2):
    return pl.pallas_call(
        _update_body,
        grid=(GRID,),
        in_specs=_COUNT_SPECS,
        out_specs=pl.BlockSpec((BLK,), lambda i: (i,)),
        out_shape=jax.ShapeDtypeStruct((P_NODES,), jnp.int32),
    )(counts3, counts3, h_idx, w_node, b_node, eps2)


def _final_body(c0_ref, c1_ref, h_ref, w_ref, b_ref, eps_ref,
                wp_ref, bp_ref, out_ref):
    idx = _new_h(c0_ref, c1_ref, h_ref, w_ref, b_ref, eps_ref)
    ii = lax.broadcasted_iota(jnp.int32, (BLK, STATE), 1)
    oh = jnp.where(ii == idx[:, None], 1.0, 0.0)
    out_ref[...] = jnp.dot(oh, wp_ref[...], preferred_element_type=jnp.float32,
                           precision=lax.Precision.HIGHEST) + bp_ref[...]


def _final_call(counts3, h_idx, w_node, b_node, eps2, w_pool, b_pool):
    return pl.pallas_call(
        _final_body,
        grid=(GRID,),
        in_specs=_COUNT_SPECS + [
            pl.BlockSpec((STATE, N_CLASSES), lambda i: (0, 0)),
            pl.BlockSpec((1, N_CLASSES), lambda i: (0, 0)),
        ],
        out_specs=pl.BlockSpec((BLK, N_CLASSES), lambda i: (i, 0)),
        out_shape=jax.ShapeDtypeStruct((P_NODES, N_CLASSES), jnp.float32),
    )(counts3, counts3, h_idx, w_node, b_node, eps2, w_pool, b_pool)


PART = 2048
NPART = EPW // PART
ZCH = 2560


def _sc_body(hidx_hbm, src_hbm, dst_hbm, t_hbm, out_hbm,
             src_v, dst_v, hidx_v, t_v, keys_v, vals_v, zb_v,
             counts_sh, sc_sem):
    cc = lax.axis_index("c")
    ss = lax.axis_index("s")

    zeros16 = jnp.zeros((16,), jnp.int32)

    def fill_z(i, carry):
        zb_v[pl.ds(i * 16, 16)] = zeros16
        return carry
    lax.fori_loop(0, ZCH // 16, fill_z, 0)

    nz = NSH // 16 // ZCH
    def zero_c(i, carry):
        pltpu.async_copy(zb_v,
                         counts_sh.at[pl.ds((ss * nz + i) * ZCH, ZCH)],
                         sc_sem)
        return carry
    lax.fori_loop(0, nz, zero_c, 0)

    base = cc * (P_EDGES // 2) + ss * EPW
    pltpu.sync_copy(hidx_hbm, hidx_v)
    pltpu.sync_copy(t_hbm, t_v)

    def zero_drain(i, carry):
        pltpu.make_async_copy(
            zb_v, counts_sh.at[pl.ds((ss * nz + i) * ZCH, ZCH)],
            sc_sem).wait()
        return carry
    lax.fori_loop(0, nz, zero_drain, 0)
    plsc.subcore_barrier()

    def part(p, carry):
        pb = base + p * PART
        pltpu.sync_copy(src_hbm.at[pl.ds(pb, PART)], src_v)
        pltpu.sync_copy(dst_hbm.at[pl.ds(pb, PART)], dst_v)

        def chunk(s, c2):
            for j in range(8):
                off = s * 128 + j * 16
                sv = src_v[pl.ds(off, 16)]
                dv = dst_v[pl.ds(off, 16)]
                spos = (sv >> 1) + (sv & 1) * HALF
                dpos = (dv >> 1) + (dv & 1) * HALF
                a = plsc.load_gather(hidx_v, [spos])
                b = plsc.load_gather(hidx_v, [dpos])
                cidx = plsc.load_gather(t_v, [a * STATE + b])
                keys_v[s, pl.ds(j * 16, 16)] = (dv >> 1) * STATE + cidx
                vals_v[s, pl.ds(j * 16, 16)] = jnp.left_shift(
                    1, (dv & 1) * 16)
            pltpu.async_copy(vals_v.at[s], counts_sh.at[keys_v.at[s]],
                             sc_sem, add=True)
            return c2
        lax.fori_loop(0, PART // 128, chunk, 0)

        def drain(s, c2):
            pltpu.make_async_copy(vals_v.at[s], counts_sh.at[keys_v.at[s]],
                                  sc_sem).wait()
            return c2
        lax.fori_loop(0, PART // 128, drain, 0)
        return carry
    lax.fori_loop(0, NPART, part, 0)
    plsc.subcore_barrier()

    och = NSH // 16
    pltpu.sync_copy(counts_sh.at[pl.ds(ss * och, och)],
                    out_hbm.at[pl.ds(cc * NSH + ss * och, och)])


@functools.lru_cache(maxsize=None)
def _get_sc_histogram():
    mesh = plsc.VectorSubcoreMesh(core_axis_name="c", subcore_axis_name="s")
    return pl.kernel(
        _sc_body,
        out_type=jax.ShapeDtypeStruct((2 * NSH,), jnp.int32),
        mesh=mesh,
        compiler_params=pltpu.CompilerParams(needs_layout_passes=False),
        scratch_types=[
            pltpu.VMEM((PART,), jnp.int32),
            pltpu.VMEM((PART,), jnp.int32),
            pltpu.VMEM((P_NODES,), jnp.int32),
            pltpu.VMEM((STATE * STATE,), jnp.int32),
            pltpu.VMEM((PART // 128, 128), jnp.int32),
            pltpu.VMEM((PART // 128, 128), jnp.int32),
            pltpu.VMEM((ZCH,), jnp.int32),
            pltpu.VMEM_SHARED((NSH,), jnp.int32),
            pltpu.SemaphoreType.DMA,
        ],
    )


def kernel(x, edge_index, W_in, b_in, W_edge, b_edge, W_node, b_node, eps,
           W_pool, b_pool):
    src = edge_index[0].astype(jnp.int32)
    dst = edge_index[1].astype(jnp.int32)
    pad = P_EDGES - N_EDGES
    srcp = jnp.concatenate([src, jnp.zeros((pad,), jnp.int32)])
    dstp = jnp.concatenate([dst, jnp.full((pad,), N_NODES, jnp.int32)])
    x_pad = jnp.concatenate(
        [x, jnp.zeros((P_NODES - N_NODES, x.shape[1]), x.dtype)])
    x_pad = x_pad[PERM]
    b_in2 = b_in.reshape(1, STATE)
    b_node2 = b_node.reshape(1, STATE)
    b_pool2 = b_pool.reshape(1, N_CLASSES)
    eps2 = eps.reshape(1, 1).astype(jnp.float32)

    h_idx = _input_call(x_pad, W_in, b_in2)
    t_flat = _table_call(W_edge, b_edge)

    sc_hist = _get_sc_histogram()
    for layer in range(NUM_LAYERS):
        counts = sc_hist(h_idx, srcp, dstp, t_flat)
        counts3 = counts.reshape(2, HALF, STATE)
        if layer < NUM_LAYERS - 1:
            h_idx = _update_call(counts3, h_idx, W_node, b_node2, eps2)
        else:
            out = _final_call(counts3, h_idx, W_node, b_node2, eps2,
                              W_pool, b_pool2)
    return out[UNPERM[:N_NODES]]


# speedup vs baseline: 1.0506x; 1.0294x over previous
import functools

import jax
import jax.numpy as jnp
import numpy as np
from jax import lax
from jax.experimental import pallas as pl
from jax.experimental.pallas import tpu as pltpu
from jax.experimental.pallas import tpu_sc as plsc

N_NODES = 10000
P_NODES = 10240
N_EDGES = 320000
N_WORKERS = 32
EPW = 10240
P_EDGES = N_WORKERS * EPW
STATE = 128
N_CLASSES = 40
NUM_LAYERS = 4
BOUND = 10
BN_SCALE = np.float32(1.0 / np.sqrt(1.0 + 1e-5))
BLK = 1024
GRID = P_NODES // BLK
HALF = P_NODES // 2
NSH = HALF * STATE
PERM = np.concatenate([np.arange(0, P_NODES, 2), np.arange(1, P_NODES, 2)])
UNPERM = np.empty((P_NODES,), np.int32)
UNPERM[PERM] = np.arange(P_NODES)


def _argmax_soft(l):
    m = jnp.max(l, axis=-1, keepdims=True)
    p = jnp.exp(l - m)
    q = p / jnp.sum(p, axis=-1, keepdims=True)
    qm = jnp.max(q, axis=-1, keepdims=True)
    ii = lax.broadcasted_iota(jnp.int32, l.shape, l.ndim - 1)
    return jnp.min(jnp.where(q == qm, ii, STATE), axis=-1)


def _input_body(x_ref, w_ref, b_ref, out_ref):
    l = jnp.dot(x_ref[...], w_ref[...], preferred_element_type=jnp.float32)
    out_ref[...] = _argmax_soft(l + b_ref[...])


def _input_call(x_pad, w_in, b_in):
    return pl.pallas_call(
        _input_body,
        grid=(GRID,),
        in_specs=[
            pl.BlockSpec((BLK, STATE), lambda i: (i, 0)),
            pl.BlockSpec((STATE, STATE), lambda i: (0, 0)),
            pl.BlockSpec((1, STATE), lambda i: (0, 0)),
        ],
        out_specs=pl.BlockSpec((BLK,), lambda i: (i,)),
        out_shape=jax.ShapeDtypeStruct((P_NODES,), jnp.int32),
    )(x_pad, w_in, b_in)


def _table_body(w_ref, b_ref, out_ref):
    i = pl.program_id(0)
    p = i * BLK + lax.broadcasted_iota(jnp.int32, (BLK, 2 * STATE), 0)
    ii = lax.broadcasted_iota(jnp.int32, (BLK, 2 * STATE), 1)
    a = p // STATE
    b = p % STATE
    e = ((ii == a) | (ii == b + STATE)).astype(jnp.float32)
    l = (jnp.dot(e, w_ref[...], preferred_element_type=jnp.float32)
         + b_ref[...]) * BN_SCALE
    out_ref[...] = _argmax_soft(l)


def _table_call(w_edge, b_edge):
    return pl.pallas_call(
        _table_body,
        grid=(STATE * STATE // BLK,),
        in_specs=[
            pl.BlockSpec((2 * STATE, STATE), lambda i: (0, 0)),
            pl.BlockSpec((1, STATE), lambda i: (0, 0)),
        ],
        out_specs=pl.BlockSpec((BLK,), lambda i: (i,)),
        out_shape=jax.ShapeDtypeStruct((STATE * STATE,), jnp.int32),
    )(w_edge, b_edge.reshape(1, STATE))


def _new_h(c0_ref, c1_ref, h_ref, w_ref, b_ref, eps_ref):
    packed = c0_ref[0] + c1_ref[0]
    lo = jnp.bitwise_and(packed, 0xFFFF)
    hi = lax.shift_right_logical(packed, 16)
    cnt = jnp.where(pl.program_id(0) < GRID // 2, lo, hi)
    aggr = jnp.minimum(cnt, BOUND).astype(jnp.float32)
    ii = lax.broadcasted_iota(jnp.int32, aggr.shape, 1)
    hcol = h_ref[...][:, None]
    a = aggr + jnp.where(ii == hcol, 1.0 + eps_ref[0, 0], 0.0)
    l = (jnp.dot(a, w_ref[...], preferred_element_type=jnp.float32)
         + b_ref[...]) * BN_SCALE
    return _argmax_soft(l)


def _update_body(c0_ref, c1_ref, h_ref, w_ref, b_ref, eps_ref, out_ref):
    out_ref[...] = _new_h(c0_ref, c1_ref, h_ref, w_ref, b_ref, eps_ref)


_COUNT_SPECS = [
    pl.BlockSpec((1, BLK, STATE), lambda i: (0, i % (GRID // 2), 0)),
    pl.BlockSpec((1, BLK, STATE), lambda i: (1, i % (GRID // 2), 0)),
    pl.BlockSpec((BLK,), lambda i: (i,)),
    pl.BlockSpec((STATE, STATE), lambda i: (0, 0)),
    pl.BlockSpec((1, STATE), lambda i: (0, 0)),
    pl.BlockSpec((1, 1), lambda i: (0, 0)),
]


def _update_call(counts3, h_idx, w_node, b_node, eps2):
    return pl.pallas_call(
        _update_body,
        grid=(GRID,),
        in_specs=_COUNT_SPECS,
        out_specs=pl.BlockSpec((BLK,), lambda i: (i,)),
        out_shape=jax.ShapeDtypeStruct((P_NODES,), jnp.int32),
    )(counts3, counts3, h_idx, w_node, b_node, eps2)


def _half_out(cnt, h, w_ref, b_ref, eps_ref, wp_ref, bp_ref):
    aggr = jnp.minimum(cnt, BOUND).astype(jnp.float32)
    ii = lax.broadcasted_iota(jnp.int32, aggr.shape, 1)
    a = aggr + jnp.where(ii == h[:, None], 1.0 + eps_ref[0, 0], 0.0)
    l = (jnp.dot(a, w_ref[...], preferred_element_type=jnp.float32)
         + b_ref[...]) * BN_SCALE
    idx = _argmax_soft(l)
    oh = jnp.where(ii == idx[:, None], 1.0, 0.0)
    return jnp.dot(oh, wp_ref[...], preferred_element_type=jnp.float32,
                   precision=lax.Precision.HIGHEST) + bp_ref[...]


def _final_body(c0_ref, c1_ref, he_ref, ho_ref, w_ref, b_ref, eps_ref,
                wp_ref, bp_ref, out_ref):
    packed = c0_ref[0] + c1_ref[0]
    out_e = _half_out(jnp.bitwise_and(packed, 0xFFFF), he_ref[...],
                      w_ref, b_ref, eps_ref, wp_ref, bp_ref)
    out_o = _half_out(lax.shift_right_logical(packed, 16), ho_ref[...],
                      w_ref, b_ref, eps_ref, wp_ref, bp_ref)
    il = jnp.stack([out_e, out_o], axis=1)
    out_ref[...] = il.reshape(BLK, N_CLASSES)


def _final_call(counts3, h_idx, w_node, b_node, eps2, w_pool, b_pool):
    hb = BLK // 2
    return pl.pallas_call(
        _final_body,
        grid=(GRID,),
        in_specs=[
            pl.BlockSpec((1, hb, STATE), lambda i: (0, i, 0)),
            pl.BlockSpec((1, hb, STATE), lambda i: (1, i, 0)),
            pl.BlockSpec((hb,), lambda i: (i,)),
            pl.BlockSpec((hb,), lambda i: (GRID + i,)),
            pl.BlockSpec((STATE, STATE), lambda i: (0, 0)),
            pl.BlockSpec((1, STATE), lambda i: (0, 0)),
            pl.BlockSpec((1, 1), lambda i: (0, 0)),
            pl.BlockSpec((STATE, N_CLASSES), lambda i: (0, 0)),
            pl.BlockSpec((1, N_CLASSES), lambda i: (0, 0)),
        ],
        out_specs=pl.BlockSpec((BLK, N_CLASSES), lambda i: (i, 0)),
        out_shape=jax.ShapeDtypeStruct((P_NODES, N_CLASSES), jnp.float32),
    )(counts3, counts3, h_idx, h_idx, w_node, b_node, eps2, w_pool, b_pool)


PART = 2048
NPART = EPW // PART
ZCH = 2560


def _sc_body(hidx_hbm, src_hbm, dst_hbm, t_hbm, out_hbm,
             src_v, dst_v, hidx_v, t_v, keys_v, vals_v, zb_v,
             counts_sh, sc_sem):
    cc = lax.axis_index("c")
    ss = lax.axis_index("s")

    zeros16 = jnp.zeros((16,), jnp.int32)

    def fill_z(i, carry):
        zb_v[pl.ds(i * 16, 16)] = zeros16
        return carry
    lax.fori_loop(0, ZCH // 16, fill_z, 0)

    nz = NSH // 16 // ZCH
    def zero_c(i, carry):
        pltpu.async_copy(zb_v,
                         counts_sh.at[pl.ds((ss * nz + i) * ZCH, ZCH)],
                         sc_sem)
        return carry
    lax.fori_loop(0, nz, zero_c, 0)

    base = cc * (P_EDGES // 2) + ss * EPW
    pltpu.sync_copy(hidx_hbm, hidx_v)
    pltpu.sync_copy(t_hbm, t_v)

    def zero_drain(i, carry):
        pltpu.make_async_copy(
            zb_v, counts_sh.at[pl.ds((ss * nz + i) * ZCH, ZCH)],
            sc_sem).wait()
        return carry
    lax.fori_loop(0, nz, zero_drain, 0)
    plsc.subcore_barrier()

    def part(p, carry):
        pb = base + p * PART
        pltpu.sync_copy(src_hbm.at[pl.ds(pb, PART)], src_v)
        pltpu.sync_copy(dst_hbm.at[pl.ds(pb, PART)], dst_v)

        def chunk(s, c2):
            for j in range(8):
                off = s * 128 + j * 16
                sv = src_v[pl.ds(off, 16)]
                dv = dst_v[pl.ds(off, 16)]
                spos = (sv >> 1) + (sv & 1) * HALF
                dpos = (dv >> 1) + (dv & 1) * HALF
                a = plsc.load_gather(hidx_v, [spos])
                b = plsc.load_gather(hidx_v, [dpos])
                cidx = plsc.load_gather(t_v, [a * STATE + b])
                keys_v[s, pl.ds(j * 16, 16)] = (dv >> 1) * STATE + cidx
                vals_v[s, pl.ds(j * 16, 16)] = jnp.left_shift(
                    1, (dv & 1) * 16)
            pltpu.async_copy(vals_v.at[s], counts_sh.at[keys_v.at[s]],
                             sc_sem, add=True)
            return c2
        lax.fori_loop(0, PART // 128, chunk, 0)

        def drain(s, c2):
            pltpu.make_async_copy(vals_v.at[s], counts_sh.at[keys_v.at[s]],
                                  sc_sem).wait()
            return c2
        lax.fori_loop(0, PART // 128, drain, 0)
        return carry
    lax.fori_loop(0, NPART, part, 0)
    plsc.subcore_barrier()

    och = NSH // 16
    pltpu.sync_copy(counts_sh.at[pl.ds(ss * och, och)],
                    out_hbm.at[pl.ds(cc * NSH + ss * och, och)])


@functools.lru_cache(maxsize=None)
def _get_sc_histogram():
    mesh = plsc.VectorSubcoreMesh(core_axis_name="c", subcore_axis_name="s")
    return pl.kernel(
        _sc_body,
        out_type=jax.ShapeDtypeStruct((2 * NSH,), jnp.int32),
        mesh=mesh,
        compiler_params=pltpu.CompilerParams(needs_layout_passes=False),
        scratch_types=[
            pltpu.VMEM((PART,), jnp.int32),
            pltpu.VMEM((PART,), jnp.int32),
            pltpu.VMEM((P_NODES,), jnp.int32),
            pltpu.VMEM((STATE * STATE,), jnp.int32),
            pltpu.VMEM((PART // 128, 128), jnp.int32),
            pltpu.VMEM((PART // 128, 128), jnp.int32),
            pltpu.VMEM((ZCH,), jnp.int32),
            pltpu.VMEM_SHARED((NSH,), jnp.int32),
            pltpu.SemaphoreType.DMA,
        ],
    )


def kernel(x, edge_index, W_in, b_in, W_edge, b_edge, W_node, b_node, eps,
           W_pool, b_pool):
    src = edge_index[0].astype(jnp.int32)
    dst = edge_index[1].astype(jnp.int32)
    pad = P_EDGES - N_EDGES
    srcp = jnp.concatenate([src, jnp.zeros((pad,), jnp.int32)])
    dstp = jnp.concatenate([dst, jnp.full((pad,), N_NODES, jnp.int32)])
    x_pad = jnp.concatenate(
        [x, jnp.zeros((P_NODES - N_NODES, x.shape[1]), x.dtype)])
    x_pad = x_pad[PERM]
    b_in2 = b_in.reshape(1, STATE)
    b_node2 = b_node.reshape(1, STATE)
    b_pool2 = b_pool.reshape(1, N_CLASSES)
    eps2 = eps.reshape(1, 1).astype(jnp.float32)

    h_idx = _input_call(x_pad, W_in, b_in2)
    t_flat = _table_call(W_edge, b_edge)

    sc_hist = _get_sc_histogram()
    for layer in range(NUM_LAYERS):
        counts = sc_hist(h_idx, srcp, dstp, t_flat)
        counts3 = counts.reshape(2, HALF, STATE)
        if layer < NUM_LAYERS - 1:
            h_idx = _update_call(counts3, h_idx, W_node, b_node2, eps2)
        else:
            out = _final_call(counts3, h_idx, W_node, b_node2, eps2,
                              W_pool, b_pool2)
    return out[:N_NODES]
